# baseline (speedup 1.0000x reference)
import functools

import jax
import jax.numpy as jnp
from jax import lax
from jax.experimental import pallas as pl
from jax.experimental.pallas import tpu as pltpu
from jax.experimental.pallas import tpu_sc as plsc

_B = 4096
_CH = 4
_L = 50
_H1 = 256
_H2 = 32
_OUT = 5
_LANES = 16
_NCHK = _H1 // _LANES
_SEG = _CH * _L


def _sc_gather_pool(x_flat, cond_flat, embed, embed_condition):
    mesh = plsc.VectorSubcoreMesh(
        core_axis_name="c", subcore_axis_name="s", num_cores=2, num_subcores=16
    )
    nw = mesh.num_cores * mesh.num_subcores
    spw = _B // nw
    gpc = 16
    idx_words = gpc * _SEG
    stage_words = gpc * _CH * _H1

    @functools.partial(
        pl.kernel,
        out_type=(
            jax.ShapeDtypeStruct((_CH * _B * _H1,), jnp.float32),
            jax.ShapeDtypeStruct((_B * _H1,), jnp.float32),
        ),
        mesh=mesh,
        scratch_types=[
            pltpu.VMEM((idx_words,), jnp.int32),
            pltpu.VMEM((_SEG, _H1), jnp.float32),
            pltpu.VMEM((_SEG, _H1), jnp.float32),
            pltpu.VMEM((stage_words,), jnp.float32),
            pltpu.SemaphoreType.DMA,
            pltpu.SemaphoreType.DMA,
        ],
    )
    def k(x_hbm, c_hbm, emb_hbm, embc_hbm, xs_out, cond_out,
          idx_v, buf_a, buf_b, stage_v, sem_a, sem_b):
        wid = lax.axis_index("s") * mesh.num_cores + lax.axis_index("c")
        b0 = wid * spw

        def gstart(table, goff, buf, sem):
            pltpu.async_copy(
                table.at[idx_v.at[pl.ds(pl.multiple_of(goff, 8), 128)]],
                buf.at[pl.ds(0, 128)], sem)
            pltpu.async_copy(
                table.at[idx_v.at[pl.ds(pl.multiple_of(goff + 128, 8),
                                        _SEG - 128)]],
                buf.at[pl.ds(128, _SEG - 128)], sem)

        def gwait(table, buf, sem):
            pltpu.make_async_copy(
                table.at[idx_v.at[pl.ds(0, 128)]],
                buf.at[pl.ds(0, 128)], sem).wait()
            pltpu.make_async_copy(
                table.at[idx_v.at[pl.ds(128, _SEG - 128)]],
                buf.at[pl.ds(128, _SEG - 128)], sem).wait()

        def accum(buf, g, stage_off):
            for s in range(4):
                init = tuple(jnp.zeros((_LANES,), jnp.float32)
                             for _ in range(_NCHK))

                @pl.loop(0, _L, init_carry=init, unroll=5)
                def acc_loop(j, acc):
                    r = s * _L + j
                    return tuple(
                        acc[c] + buf[r, pl.ds(c * _LANES, _LANES)]
                        for c in range(_NCHK))

                off = stage_off(g, s)
                for c in range(_NCHK):
                    stage_v[pl.ds(off + c * _LANES, _LANES)] = acc_loop[c]

        def run_phase(table, idx_hbm, n_chunks, idx_off, stage_off, write_fn):
            @pl.loop(0, n_chunks)
            def _chunk(ci):
                pltpu.sync_copy(
                    idx_hbm.at[pl.ds(pl.multiple_of(idx_off(ci), 8),
                                     idx_words)], idx_v)
                gstart(table, 0, buf_a, sem_a)

                @pl.loop(0, gpc, step=2)
                def _g(g):
                    gstart(table, (g + 1) * _SEG, buf_b, sem_b)
                    gwait(table, buf_a, sem_a)
                    accum(buf_a, g, stage_off)

                    @pl.when(g < gpc - 2)
                    def _():
                        gstart(table, (g + 2) * _SEG, buf_a, sem_a)

                    gwait(table, buf_b, sem_b)
                    accum(buf_b, g + 1, stage_off)

                write_fn(ci)

        def xs_write(ci):
            cb = b0 + ci * gpc
            for ch in range(_CH):
                pltpu.sync_copy(
                    stage_v.at[pl.ds(ch * gpc * _H1, gpc * _H1)],
                    xs_out.at[pl.ds(pl.multiple_of((ch * _B + cb) * _H1, 8),
                                    gpc * _H1)])

        run_phase(
            emb_hbm, x_hbm, spw // gpc,
            lambda ci: (b0 + ci * gpc) * _SEG,
            lambda g, s: s * (gpc * _H1) + g * _H1,
            xs_write)

        def cond_write(ci):
            cb = b0 + ci * 4 * gpc
            pltpu.sync_copy(
                stage_v,
                cond_out.at[pl.ds(pl.multiple_of(cb * _H1, 8), stage_words)])

        run_phase(
            embc_hbm, c_hbm, spw // (4 * gpc),
            lambda ci: (b0 + ci * 4 * gpc) * _L,
            lambda g, s: (g * 4 + s) * _H1,
            cond_write)

    return k(x_flat, cond_flat, embed, embed_condition)


def _tc_mlp(xs_raw, cond_raw, Wc, bc, W2, b2, W3, b3, W4, b4):
    bb = 512
    dn = (((1,), (1,)), ((), ()))

    def body(xs_ref, c_ref, wc_ref, bc_ref, w2_ref, b2_ref, w3_ref, b3_ref,
             w4_ref, b4_ref, out_ref):
        xs = jnp.maximum(xs_ref[...], 0.0)
        xsum = xs[0] + xs[1] + xs[2] + xs[3]
        c = jnp.maximum(c_ref[...], 0.0) + xsum
        c2 = lax.dot_general(c, wc_ref[...], dn,
                             preferred_element_type=jnp.float32)
        c2 = jnp.maximum(c2 + bc_ref[...], 0.0)
        outs = []
        for ch in range(_CH):
            h = lax.dot_general(xs[ch], w2_ref[...], dn,
                                preferred_element_type=jnp.float32)
            h = jnp.maximum(h + b2_ref[...] + c2, 0.0)
            h = lax.dot_general(h, w3_ref[...], dn,
                                preferred_element_type=jnp.float32)
            h = jnp.maximum(h + b3_ref[...], 0.0)
            o = lax.dot_general(h, w4_ref[...], dn,
                                preferred_element_type=jnp.float32)
            outs.append(o + b4_ref[...])
        out_ref[...] = jnp.concatenate(outs, axis=1)

    full = lambda shape: pl.BlockSpec(shape, lambda i: tuple(0 for _ in shape))
    return pl.pallas_call(
        body,
        grid=(_B // bb,),
        in_specs=[
            pl.BlockSpec((_CH, bb, _H1), lambda i: (0, i, 0)),
            pl.BlockSpec((bb, _H1), lambda i: (i, 0)),
            full((_H2, _H1)),
            full((1, _H2)),
            full((_H2, _H1)),
            full((1, _H2)),
            full((_H2, _H2)),
            full((1, _H2)),
            full((_OUT, _H2)),
            full((1, _OUT)),
        ],
        out_specs=pl.BlockSpec((bb, _CH * _OUT), lambda i: (i, 0)),
        out_shape=jax.ShapeDtypeStruct((_B, _CH * _OUT), jnp.float32),
    )(xs_raw, cond_raw, Wc, bc.reshape(1, _H2), W2, b2.reshape(1, _H2),
      W3, b3.reshape(1, _H2), W4, b4.reshape(1, _OUT))


def kernel(x, condition, embed, embed_condition, Wc, bc, W2, b2, W3, b3,
           W4, b4):
    x_flat = x.reshape(-1).astype(jnp.int32)
    cond_flat = condition.reshape(-1).astype(jnp.int32)
    xs_flat, cond_pool = _sc_gather_pool(x_flat, cond_flat, embed,
                                         embed_condition)
    xs_raw = xs_flat.reshape(_CH, _B, _H1)
    cond_raw = cond_pool.reshape(_B, _H1)
    out = _tc_mlp(xs_raw, cond_raw, Wc, bc, W2, b2, W3, b3, W4, b4)
    return out.reshape(_B, _CH, _OUT)


# speedup vs baseline: 1.0523x; 1.0523x over previous
import functools

import jax
import jax.numpy as jnp
from jax import lax
from jax.experimental import pallas as pl
from jax.experimental.pallas import tpu as pltpu
from jax.experimental.pallas import tpu_sc as plsc

_B = 4096
_CH = 4
_L = 50
_H1 = 256
_H2 = 32
_OUT = 5
_LANES = 16
_NCHK = _H1 // _LANES
_SEG = _CH * _L


def _sc_gather_pool(x_flat, cond_flat, embed, embed_condition):
    mesh = plsc.VectorSubcoreMesh(
        core_axis_name="c", subcore_axis_name="s", num_cores=2, num_subcores=16
    )
    nw = mesh.num_cores * mesh.num_subcores
    spw = _B // nw
    gpc = 16
    idx_words = gpc * _SEG
    stage_words = gpc * _CH * _H1

    @functools.partial(
        pl.kernel,
        out_type=(
            jax.ShapeDtypeStruct((_CH * _B * _H1,), jnp.float32),
            jax.ShapeDtypeStruct((_B * _H1,), jnp.float32),
        ),
        mesh=mesh,
        scratch_types=[
            pltpu.VMEM((idx_words,), jnp.int32),
            pltpu.VMEM((_SEG, _H1), jnp.float32),
            pltpu.VMEM((_SEG, _H1), jnp.float32),
            pltpu.VMEM((stage_words,), jnp.float32),
            pltpu.SemaphoreType.DMA,
            pltpu.SemaphoreType.DMA,
        ],
    )
    def k(x_hbm, c_hbm, emb_hbm, embc_hbm, xs_out, cond_out,
          idx_v, buf_a, buf_b, stage_v, sem_a, sem_b):
        wid = lax.axis_index("s") * mesh.num_cores + lax.axis_index("c")
        b0 = wid * spw

        def gstart(table, goff, buf, sem):
            pltpu.async_copy(
                table.at[idx_v.at[pl.ds(pl.multiple_of(goff, 8), 128)]],
                buf.at[pl.ds(0, 128)], sem)
            pltpu.async_copy(
                table.at[idx_v.at[pl.ds(pl.multiple_of(goff + 128, 8),
                                        _SEG - 128)]],
                buf.at[pl.ds(128, _SEG - 128)], sem)

        def gwait(table, buf, sem):
            pltpu.make_async_copy(
                table.at[idx_v.at[pl.ds(0, 128)]],
                buf.at[pl.ds(0, 128)], sem).wait()
            pltpu.make_async_copy(
                table.at[idx_v.at[pl.ds(128, _SEG - 128)]],
                buf.at[pl.ds(128, _SEG - 128)], sem).wait()

        def accum(buf, g, stage_off):
            for s in range(4):
                init = tuple(jnp.zeros((_LANES,), jnp.float32)
                             for _ in range(_NCHK))

                @pl.loop(0, _L, init_carry=init, unroll=2)
                def acc_loop(j, acc):
                    r = s * _L + j
                    return tuple(
                        acc[c] + buf[r, pl.ds(c * _LANES, _LANES)]
                        for c in range(_NCHK))

                off = stage_off(g, s)
                for c in range(_NCHK):
                    stage_v[pl.ds(off + c * _LANES, _LANES)] = acc_loop[c]

        def run_phase(table, idx_hbm, n_chunks, idx_off, stage_off, write_fn):
            @pl.loop(0, n_chunks)
            def _chunk(ci):
                pltpu.sync_copy(
                    idx_hbm.at[pl.ds(pl.multiple_of(idx_off(ci), 8),
                                     idx_words)], idx_v)
                gstart(table, 0, buf_a, sem_a)

                @pl.loop(0, gpc, step=2)
                def _g(g):
                    gstart(table, (g + 1) * _SEG, buf_b, sem_b)
                    gwait(table, buf_a, sem_a)
                    accum(buf_a, g, stage_off)

                    @pl.when(g < gpc - 2)
                    def _():
                        gstart(table, (g + 2) * _SEG, buf_a, sem_a)

                    gwait(table, buf_b, sem_b)
                    accum(buf_b, g + 1, stage_off)

                write_fn(ci)

        def xs_write(ci):
            cb = b0 + ci * gpc
            for ch in range(_CH):
                pltpu.sync_copy(
                    stage_v.at[pl.ds(ch * gpc * _H1, gpc * _H1)],
                    xs_out.at[pl.ds(pl.multiple_of((ch * _B + cb) * _H1, 8),
                                    gpc * _H1)])

        run_phase(
            emb_hbm, x_hbm, spw // gpc,
            lambda ci: (b0 + ci * gpc) * _SEG,
            lambda g, s: s * (gpc * _H1) + g * _H1,
            xs_write)

        def cond_write(ci):
            cb = b0 + ci * 4 * gpc
            pltpu.sync_copy(
                stage_v,
                cond_out.at[pl.ds(pl.multiple_of(cb * _H1, 8), stage_words)])

        run_phase(
            embc_hbm, c_hbm, spw // (4 * gpc),
            lambda ci: (b0 + ci * 4 * gpc) * _L,
            lambda g, s: (g * 4 + s) * _H1,
            cond_write)

    return k(x_flat, cond_flat, embed, embed_condition)


def _tc_mlp(xs_raw, cond_raw, Wc, bc, W2, b2, W3, b3, W4, b4):
    bb = 512
    dn = (((1,), (1,)), ((), ()))

    def body(xs_ref, c_ref, wc_ref, bc_ref, w2_ref, b2_ref, w3_ref, b3_ref,
             w4_ref, b4_ref, out_ref):
        xs = jnp.maximum(xs_ref[...], 0.0)
        xsum = xs[0] + xs[1] + xs[2] + xs[3]
        c = jnp.maximum(c_ref[...], 0.0) + xsum
        c2 = lax.dot_general(c, wc_ref[...], dn,
                             preferred_element_type=jnp.float32)
        c2 = jnp.maximum(c2 + bc_ref[...], 0.0)
        outs = []
        for ch in range(_CH):
            h = lax.dot_general(xs[ch], w2_ref[...], dn,
                                preferred_element_type=jnp.float32)
            h = jnp.maximum(h + b2_ref[...] + c2, 0.0)
            h = lax.dot_general(h, w3_ref[...], dn,
                                preferred_element_type=jnp.float32)
            h = jnp.maximum(h + b3_ref[...], 0.0)
            o = lax.dot_general(h, w4_ref[...], dn,
                                preferred_element_type=jnp.float32)
            outs.append(o + b4_ref[...])
        out_ref[...] = jnp.concatenate(outs, axis=1)

    full = lambda shape: pl.BlockSpec(shape, lambda i: tuple(0 for _ in shape))
    return pl.pallas_call(
        body,
        grid=(_B // bb,),
        in_specs=[
            pl.BlockSpec((_CH, bb, _H1), lambda i: (0, i, 0)),
            pl.BlockSpec((bb, _H1), lambda i: (i, 0)),
            full((_H2, _H1)),
            full((1, _H2)),
            full((_H2, _H1)),
            full((1, _H2)),
            full((_H2, _H2)),
            full((1, _H2)),
            full((_OUT, _H2)),
            full((1, _OUT)),
        ],
        out_specs=pl.BlockSpec((bb, _CH * _OUT), lambda i: (i, 0)),
        out_shape=jax.ShapeDtypeStruct((_B, _CH * _OUT), jnp.float32),
    )(xs_raw, cond_raw, Wc, bc.reshape(1, _H2), W2, b2.reshape(1, _H2),
      W3, b3.reshape(1, _H2), W4, b4.reshape(1, _OUT))


def kernel(x, condition, embed, embed_condition, Wc, bc, W2, b2, W3, b3,
           W4, b4):
    x_flat = x.reshape(-1).astype(jnp.int32)
    cond_flat = condition.reshape(-1).astype(jnp.int32)
    xs_flat, cond_pool = _sc_gather_pool(x_flat, cond_flat, embed,
                                         embed_condition)
    xs_raw = xs_flat.reshape(_CH, _B, _H1)
    cond_raw = cond_pool.reshape(_B, _H1)
    out = _tc_mlp(xs_raw, cond_raw, Wc, bc, W2, b2, W3, b3, W4, b4)
    return out.reshape(_B, _CH, _OUT)


# speedup vs baseline: 1.1294x; 1.0732x over previous
import functools

import jax
import jax.numpy as jnp
from jax import lax
from jax.experimental import pallas as pl
from jax.experimental.pallas import tpu as pltpu
from jax.experimental.pallas import tpu_sc as plsc

_B = 4096
_CH = 4
_L = 50
_H1 = 256
_H2 = 32
_OUT = 5
_LANES = 16
_NCHK = _H1 // _LANES
_SEG = _CH * _L


def _sc_gather_pool(x_flat, cond_flat, embed, embed_condition):
    mesh = plsc.VectorSubcoreMesh(
        core_axis_name="c", subcore_axis_name="s", num_cores=2, num_subcores=16
    )
    nw = mesh.num_cores * mesh.num_subcores
    spw = _B // nw
    gpc = 16
    idx_words = gpc * _SEG
    stage_words = gpc * _CH * _H1

    @functools.partial(
        pl.kernel,
        out_type=(
            jax.ShapeDtypeStruct((_CH * _B * _H1,), jnp.float32),
            jax.ShapeDtypeStruct((_B * _H1,), jnp.float32),
        ),
        mesh=mesh,
        scratch_types=[
            pltpu.VMEM((idx_words,), jnp.int32),
            pltpu.VMEM((idx_words,), jnp.int32),
            pltpu.VMEM((_SEG, _H1), jnp.float32),
            pltpu.VMEM((_SEG, _H1), jnp.float32),
            pltpu.VMEM((stage_words,), jnp.float32),
            pltpu.SemaphoreType.DMA,
            pltpu.SemaphoreType.DMA,
            pltpu.SemaphoreType.DMA,
        ],
    )
    def k(x_hbm, c_hbm, emb_hbm, embc_hbm, xs_out, cond_out,
          idx_a, idx_b, buf_a, buf_b, stage_v, sem_a, sem_b, sem_i):
        wid = lax.axis_index("s") * mesh.num_cores + lax.axis_index("c")
        b0 = wid * spw

        def gstart(table, idx_v, goff, buf, sem):
            pltpu.async_copy(
                table.at[idx_v.at[pl.ds(pl.multiple_of(goff, 8), 128)]],
                buf.at[pl.ds(0, 128)], sem)
            pltpu.async_copy(
                table.at[idx_v.at[pl.ds(pl.multiple_of(goff + 128, 8),
                                        _SEG - 128)]],
                buf.at[pl.ds(128, _SEG - 128)], sem)

        def gwait(table, buf, sem):
            pltpu.make_async_copy(
                table.at[idx_a.at[pl.ds(0, 128)]],
                buf.at[pl.ds(0, 128)], sem).wait()
            pltpu.make_async_copy(
                table.at[idx_a.at[pl.ds(128, _SEG - 128)]],
                buf.at[pl.ds(128, _SEG - 128)], sem).wait()

        def accum(buf, g, stage_off):
            for s in range(4):
                init = tuple(jnp.zeros((_LANES,), jnp.float32)
                             for _ in range(_NCHK))

                @pl.loop(0, _L, init_carry=init, unroll=2)
                def acc_loop(j, acc):
                    r = s * _L + j
                    return tuple(
                        acc[c] + buf[r, pl.ds(c * _LANES, _LANES)]
                        for c in range(_NCHK))

                off = stage_off(g, s)
                for c in range(_NCHK):
                    stage_v[pl.ds(off + c * _LANES, _LANES)] = acc_loop[c]

        def run_phase(table, idx_hbm, n_chunks, idx_off, stage_off, write_fn):
            def chunk_body(ci, cur_idx, nxt_idx):
                @pl.when(ci + 1 < n_chunks)
                def _():
                    pltpu.async_copy(
                        idx_hbm.at[pl.ds(pl.multiple_of(idx_off(ci + 1), 8),
                                         idx_words)], nxt_idx, sem_i)

                @pl.loop(0, gpc - 2, step=2)
                def _g(j):
                    gwait(table, buf_a, sem_a)
                    accum(buf_a, j, stage_off)
                    gstart(table, cur_idx, (j + 2) * _SEG, buf_a, sem_a)
                    gwait(table, buf_b, sem_b)
                    accum(buf_b, j + 1, stage_off)
                    gstart(table, cur_idx, (j + 3) * _SEG, buf_b, sem_b)

                gwait(table, buf_a, sem_a)
                accum(buf_a, gpc - 2, stage_off)

                @pl.when(ci + 1 < n_chunks)
                def _():
                    pltpu.make_async_copy(
                        idx_hbm.at[pl.ds(0, idx_words)], nxt_idx,
                        sem_i).wait()
                    gstart(table, nxt_idx, 0, buf_a, sem_a)

                gwait(table, buf_b, sem_b)
                accum(buf_b, gpc - 1, stage_off)

                @pl.when(ci + 1 < n_chunks)
                def _():
                    gstart(table, nxt_idx, _SEG, buf_b, sem_b)

                write_fn(ci)

            pltpu.sync_copy(
                idx_hbm.at[pl.ds(pl.multiple_of(idx_off(0), 8), idx_words)],
                idx_a)
            gstart(table, idx_a, 0, buf_a, sem_a)
            gstart(table, idx_a, _SEG, buf_b, sem_b)

            @pl.loop(0, n_chunks, step=2)
            def _cpair(ci):
                chunk_body(ci, idx_a, idx_b)
                chunk_body(ci + 1, idx_b, idx_a)

        def xs_write(ci):
            cb = b0 + ci * gpc
            for ch in range(_CH):
                pltpu.sync_copy(
                    stage_v.at[pl.ds(ch * gpc * _H1, gpc * _H1)],
                    xs_out.at[pl.ds(pl.multiple_of((ch * _B + cb) * _H1, 8),
                                    gpc * _H1)])

        run_phase(
            emb_hbm, x_hbm, spw // gpc,
            lambda ci: (b0 + ci * gpc) * _SEG,
            lambda g, s: s * (gpc * _H1) + g * _H1,
            xs_write)

        def cond_write(ci):
            cb = b0 + ci * 4 * gpc
            pltpu.sync_copy(
                stage_v,
                cond_out.at[pl.ds(pl.multiple_of(cb * _H1, 8), stage_words)])

        run_phase(
            embc_hbm, c_hbm, spw // (4 * gpc),
            lambda ci: (b0 + ci * 4 * gpc) * _L,
            lambda g, s: (g * 4 + s) * _H1,
            cond_write)

    return k(x_flat, cond_flat, embed, embed_condition)


def _tc_mlp(xs_raw, cond_raw, Wc, bc, W2, b2, W3, b3, W4, b4):
    bb = 512
    dn = (((1,), (1,)), ((), ()))

    def body(xs_ref, c_ref, wc_ref, bc_ref, w2_ref, b2_ref, w3_ref, b3_ref,
             w4_ref, b4_ref, out_ref):
        xs = jnp.maximum(xs_ref[...], 0.0)
        xsum = xs[0] + xs[1] + xs[2] + xs[3]
        c = jnp.maximum(c_ref[...], 0.0) + xsum
        c2 = lax.dot_general(c, wc_ref[...], dn,
                             preferred_element_type=jnp.float32)
        c2 = jnp.maximum(c2 + bc_ref[...], 0.0)
        outs = []
        for ch in range(_CH):
            h = lax.dot_general(xs[ch], w2_ref[...], dn,
                                preferred_element_type=jnp.float32)
            h = jnp.maximum(h + b2_ref[...] + c2, 0.0)
            h = lax.dot_general(h, w3_ref[...], dn,
                                preferred_element_type=jnp.float32)
            h = jnp.maximum(h + b3_ref[...], 0.0)
            o = lax.dot_general(h, w4_ref[...], dn,
                                preferred_element_type=jnp.float32)
            outs.append(o + b4_ref[...])
        out_ref[...] = jnp.concatenate(outs, axis=1)

    full = lambda shape: pl.BlockSpec(shape, lambda i: tuple(0 for _ in shape))
    return pl.pallas_call(
        body,
        grid=(_B // bb,),
        in_specs=[
            pl.BlockSpec((_CH, bb, _H1), lambda i: (0, i, 0)),
            pl.BlockSpec((bb, _H1), lambda i: (i, 0)),
            full((_H2, _H1)),
            full((1, _H2)),
            full((_H2, _H1)),
            full((1, _H2)),
            full((_H2, _H2)),
            full((1, _H2)),
            full((_OUT, _H2)),
            full((1, _OUT)),
        ],
        out_specs=pl.BlockSpec((bb, _CH * _OUT), lambda i: (i, 0)),
        out_shape=jax.ShapeDtypeStruct((_B, _CH * _OUT), jnp.float32),
    )(xs_raw, cond_raw, Wc, bc.reshape(1, _H2), W2, b2.reshape(1, _H2),
      W3, b3.reshape(1, _H2), W4, b4.reshape(1, _OUT))


def kernel(x, condition, embed, embed_condition, Wc, bc, W2, b2, W3, b3,
           W4, b4):
    x_flat = x.reshape(-1).astype(jnp.int32)
    cond_flat = condition.reshape(-1).astype(jnp.int32)
    xs_flat, cond_pool = _sc_gather_pool(x_flat, cond_flat, embed,
                                         embed_condition)
    xs_raw = xs_flat.reshape(_CH, _B, _H1)
    cond_raw = cond_pool.reshape(_B, _H1)
    out = _tc_mlp(xs_raw, cond_raw, Wc, bc, W2, b2, W3, b3, W4, b4)
    return out.reshape(_B, _CH, _OUT)


# speedup vs baseline: 1.2138x; 1.0748x over previous
import functools

import jax
import jax.numpy as jnp
from jax import lax
from jax.experimental import pallas as pl
from jax.experimental.pallas import tpu as pltpu
from jax.experimental.pallas import tpu_sc as plsc

_B = 4096
_CH = 4
_L = 50
_H1 = 256
_H2 = 32
_OUT = 5
_LANES = 16
_NCHK = _H1 // _LANES
_SEG = _CH * _L


def _sc_gather_pool(x_flat, cond_flat, embed, embed_condition):
    mesh = plsc.VectorSubcoreMesh(
        core_axis_name="c", subcore_axis_name="s", num_cores=2, num_subcores=16
    )
    nw = mesh.num_cores * mesh.num_subcores
    spw = _B // nw
    gpc = 16
    idx_words = gpc * _SEG
    stage_words = gpc * _CH * _H1

    @functools.partial(
        pl.kernel,
        out_type=(
            jax.ShapeDtypeStruct((_CH * _B * _H1,), jnp.float32),
            jax.ShapeDtypeStruct((_B * _H1,), jnp.float32),
        ),
        mesh=mesh,
        scratch_types=[
            pltpu.VMEM((idx_words,), jnp.int32),
            pltpu.VMEM((idx_words,), jnp.int32),
            pltpu.VMEM((_SEG, _H1), jnp.float32),
            pltpu.VMEM((_SEG, _H1), jnp.float32),
            pltpu.VMEM((stage_words,), jnp.float32),
            pltpu.SemaphoreType.DMA,
            pltpu.SemaphoreType.DMA,
            pltpu.SemaphoreType.DMA,
        ],
    )
    def k(x_hbm, c_hbm, emb_hbm, embc_hbm, xs_out, cond_out,
          idx_a, idx_b, buf_a, buf_b, stage_v, sem_a, sem_b, sem_i):
        wid = lax.axis_index("s") * mesh.num_cores + lax.axis_index("c")
        b0 = wid * spw

        def gstart(table, idx_v, goff, buf, sem):
            pltpu.async_copy(
                table.at[idx_v.at[pl.ds(pl.multiple_of(goff, 8), 128)]],
                buf.at[pl.ds(0, 128)], sem)
            pltpu.async_copy(
                table.at[idx_v.at[pl.ds(pl.multiple_of(goff + 128, 8),
                                        _SEG - 128)]],
                buf.at[pl.ds(128, _SEG - 128)], sem)

        def gwait1(table, buf, sem):
            pltpu.make_async_copy(
                table.at[idx_a.at[pl.ds(0, 128)]],
                buf.at[pl.ds(0, 128)], sem).wait()

        def gwait2(table, buf, sem):
            pltpu.make_async_copy(
                table.at[idx_a.at[pl.ds(128, _SEG - 128)]],
                buf.at[pl.ds(128, _SEG - 128)], sem).wait()

        def accum(buf, g, stage_off, segs):
            for s in segs:
                init = tuple(jnp.zeros((_LANES,), jnp.float32)
                             for _ in range(_NCHK))

                @pl.loop(0, _L, init_carry=init, unroll=2)
                def acc_loop(j, acc):
                    r = s * _L + j
                    return tuple(
                        acc[c] + buf[r, pl.ds(c * _LANES, _LANES)]
                        for c in range(_NCHK))

                off = stage_off(g, s)
                for c in range(_NCHK):
                    stage_v[pl.ds(off + c * _LANES, _LANES)] = acc_loop[c]

        def process(table, buf, sem, j, stage_off):
            gwait1(table, buf, sem)
            accum(buf, j, stage_off, (0, 1))
            gwait2(table, buf, sem)
            accum(buf, j, stage_off, (2, 3))

        def run_phase(table, idx_hbm, n_chunks, idx_off, stage_off, write_fn):
            def chunk_body(ci, cur_idx, nxt_idx):
                @pl.when(ci + 1 < n_chunks)
                def _():
                    pltpu.async_copy(
                        idx_hbm.at[pl.ds(pl.multiple_of(idx_off(ci + 1), 8),
                                         idx_words)], nxt_idx, sem_i)

                @pl.loop(0, gpc - 2, step=2)
                def _g(j):
                    process(table, buf_a, sem_a, j, stage_off)
                    gstart(table, cur_idx, (j + 2) * _SEG, buf_a, sem_a)
                    process(table, buf_b, sem_b, j + 1, stage_off)
                    gstart(table, cur_idx, (j + 3) * _SEG, buf_b, sem_b)

                process(table, buf_a, sem_a, gpc - 2, stage_off)

                @pl.when(ci + 1 < n_chunks)
                def _():
                    pltpu.make_async_copy(
                        idx_hbm.at[pl.ds(0, idx_words)], nxt_idx,
                        sem_i).wait()
                    gstart(table, nxt_idx, 0, buf_a, sem_a)

                process(table, buf_b, sem_b, gpc - 1, stage_off)

                @pl.when(ci + 1 < n_chunks)
                def _():
                    gstart(table, nxt_idx, _SEG, buf_b, sem_b)

                write_fn(ci)

            pltpu.sync_copy(
                idx_hbm.at[pl.ds(pl.multiple_of(idx_off(0), 8), idx_words)],
                idx_a)
            gstart(table, idx_a, 0, buf_a, sem_a)
            gstart(table, idx_a, _SEG, buf_b, sem_b)

            @pl.loop(0, n_chunks, step=2)
            def _cpair(ci):
                chunk_body(ci, idx_a, idx_b)
                chunk_body(ci + 1, idx_b, idx_a)

        def xs_write(ci):
            cb = b0 + ci * gpc
            for ch in range(_CH):
                pltpu.sync_copy(
                    stage_v.at[pl.ds(ch * gpc * _H1, gpc * _H1)],
                    xs_out.at[pl.ds(pl.multiple_of((ch * _B + cb) * _H1, 8),
                                    gpc * _H1)])

        run_phase(
            emb_hbm, x_hbm, spw // gpc,
            lambda ci: (b0 + ci * gpc) * _SEG,
            lambda g, s: s * (gpc * _H1) + g * _H1,
            xs_write)

        def cond_write(ci):
            cb = b0 + ci * 4 * gpc
            pltpu.sync_copy(
                stage_v,
                cond_out.at[pl.ds(pl.multiple_of(cb * _H1, 8), stage_words)])

        run_phase(
            embc_hbm, c_hbm, spw // (4 * gpc),
            lambda ci: (b0 + ci * 4 * gpc) * _L,
            lambda g, s: (g * 4 + s) * _H1,
            cond_write)

    return k(x_flat, cond_flat, embed, embed_condition)


def _tc_mlp(xs_raw, cond_raw, Wc, bc, W2, b2, W3, b3, W4, b4):
    bb = 512
    dn = (((1,), (1,)), ((), ()))

    def body(xs_ref, c_ref, wc_ref, bc_ref, w2_ref, b2_ref, w3_ref, b3_ref,
             w4_ref, b4_ref, out_ref):
        xs = jnp.maximum(xs_ref[...], 0.0)
        xsum = xs[0] + xs[1] + xs[2] + xs[3]
        c = jnp.maximum(c_ref[...], 0.0) + xsum
        c2 = lax.dot_general(c, wc_ref[...], dn,
                             preferred_element_type=jnp.float32)
        c2 = jnp.maximum(c2 + bc_ref[...], 0.0)
        outs = []
        for ch in range(_CH):
            h = lax.dot_general(xs[ch], w2_ref[...], dn,
                                preferred_element_type=jnp.float32)
            h = jnp.maximum(h + b2_ref[...] + c2, 0.0)
            h = lax.dot_general(h, w3_ref[...], dn,
                                preferred_element_type=jnp.float32)
            h = jnp.maximum(h + b3_ref[...], 0.0)
            o = lax.dot_general(h, w4_ref[...], dn,
                                preferred_element_type=jnp.float32)
            outs.append(o + b4_ref[...])
        out_ref[...] = jnp.concatenate(outs, axis=1)

    full = lambda shape: pl.BlockSpec(shape, lambda i: tuple(0 for _ in shape))
    return pl.pallas_call(
        body,
        grid=(_B // bb,),
        in_specs=[
            pl.BlockSpec((_CH, bb, _H1), lambda i: (0, i, 0)),
            pl.BlockSpec((bb, _H1), lambda i: (i, 0)),
            full((_H2, _H1)),
            full((1, _H2)),
            full((_H2, _H1)),
            full((1, _H2)),
            full((_H2, _H2)),
            full((1, _H2)),
            full((_OUT, _H2)),
            full((1, _OUT)),
        ],
        out_specs=pl.BlockSpec((bb, _CH * _OUT), lambda i: (i, 0)),
        out_shape=jax.ShapeDtypeStruct((_B, _CH * _OUT), jnp.float32),
    )(xs_raw, cond_raw, Wc, bc.reshape(1, _H2), W2, b2.reshape(1, _H2),
      W3, b3.reshape(1, _H2), W4, b4.reshape(1, _OUT))


def kernel(x, condition, embed, embed_condition, Wc, bc, W2, b2, W3, b3,
           W4, b4):
    x_flat = x.reshape(-1).astype(jnp.int32)
    cond_flat = condition.reshape(-1).astype(jnp.int32)
    xs_flat, cond_pool = _sc_gather_pool(x_flat, cond_flat, embed,
                                         embed_condition)
    xs_raw = xs_flat.reshape(_CH, _B, _H1)
    cond_raw = cond_pool.reshape(_B, _H1)
    out = _tc_mlp(xs_raw, cond_raw, Wc, bc, W2, b2, W3, b3, W4, b4)
    return out.reshape(_B, _CH, _OUT)


# speedup vs baseline: 1.2229x; 1.0074x over previous
import functools

import jax
import jax.numpy as jnp
from jax import lax
from jax.experimental import pallas as pl
from jax.experimental.pallas import tpu as pltpu
from jax.experimental.pallas import tpu_sc as plsc

_B = 4096
_CH = 4
_L = 50
_H1 = 256
_H2 = 32
_OUT = 5
_LANES = 16
_NCHK = _H1 // _LANES
_SEG = _CH * _L


def _sc_gather_pool(x_flat, cond_flat, embed, embed_condition):
    mesh = plsc.VectorSubcoreMesh(
        core_axis_name="c", subcore_axis_name="s", num_cores=2, num_subcores=16
    )
    nw = mesh.num_cores * mesh.num_subcores
    spw = _B // nw
    gpc = 16
    idx_words = gpc * _SEG
    stage_words = gpc * _CH * _H1

    @functools.partial(
        pl.kernel,
        out_type=(
            jax.ShapeDtypeStruct((_CH * _B * _H1,), jnp.float32),
            jax.ShapeDtypeStruct((_B * _H1,), jnp.float32),
        ),
        mesh=mesh,
        scratch_types=[
            pltpu.VMEM((idx_words,), jnp.int32),
            pltpu.VMEM((idx_words,), jnp.int32),
            pltpu.VMEM((128, _H1), jnp.float32),
            pltpu.VMEM((_SEG - 128, _H1), jnp.float32),
            pltpu.VMEM((128, _H1), jnp.float32),
            pltpu.VMEM((_SEG - 128, _H1), jnp.float32),
            pltpu.VMEM((stage_words,), jnp.float32),
            pltpu.SemaphoreType.DMA,
            pltpu.SemaphoreType.DMA,
            pltpu.SemaphoreType.DMA,
            pltpu.SemaphoreType.DMA,
            pltpu.SemaphoreType.DMA,
        ],
    )
    def k(x_hbm, c_hbm, emb_hbm, embc_hbm, xs_out, cond_out,
          idx_a, idx_b, buf_a1, buf_a2, buf_b1, buf_b2, stage_v,
          sem_a1, sem_a2, sem_b1, sem_b2, sem_i):
        wid = lax.axis_index("s") * mesh.num_cores + lax.axis_index("c")
        b0 = wid * spw

        def gstart1(table, idx_v, goff, b1, s1):
            pltpu.async_copy(
                table.at[idx_v.at[pl.ds(pl.multiple_of(goff, 8), 128)]],
                b1, s1)

        def gstart2(table, idx_v, goff, b2, s2):
            pltpu.async_copy(
                table.at[idx_v.at[pl.ds(pl.multiple_of(goff + 128, 8),
                                        _SEG - 128)]], b2, s2)

        def gwait1(table, b1, s1):
            pltpu.make_async_copy(
                table.at[idx_a.at[pl.ds(0, 128)]], b1, s1).wait()

        def gwait2(table, b2, s2):
            pltpu.make_async_copy(
                table.at[idx_a.at[pl.ds(128, _SEG - 128)]], b2, s2).wait()

        def accum_span(buf, row0, n, soff, cont):
            if cont:
                init = tuple(stage_v[pl.ds(soff + c * _LANES, _LANES)]
                             for c in range(_NCHK))
            else:
                init = tuple(jnp.zeros((_LANES,), jnp.float32)
                             for _ in range(_NCHK))

            @pl.loop(0, n, init_carry=init, unroll=2)
            def acc_loop(j, acc):
                r = row0 + j
                return tuple(acc[c] + buf[r, pl.ds(c * _LANES, _LANES)]
                             for c in range(_NCHK))

            for c in range(_NCHK):
                stage_v[pl.ds(soff + c * _LANES, _LANES)] = acc_loop[c]

        def process(table, b1, b2, s1, s2, g, stage_off, refill1, refill2):
            gwait1(table, b1, s1)
            accum_span(b1, 0, _L, stage_off(g, 0), False)
            accum_span(b1, _L, _L, stage_off(g, 1), False)
            accum_span(b1, 2 * _L, 128 - 2 * _L, stage_off(g, 2), False)
            refill1()
            gwait2(table, b2, s2)
            accum_span(b2, 0, 3 * _L - 128, stage_off(g, 2), True)
            accum_span(b2, 3 * _L - 128, _L, stage_off(g, 3), False)
            refill2()

        def run_phase(table, idx_hbm, n_chunks, idx_off, stage_off, write_fn):
            def chunk_body(ci, cur_idx, nxt_idx):
                @pl.when(ci + 1 < n_chunks)
                def _():
                    pltpu.async_copy(
                        idx_hbm.at[pl.ds(pl.multiple_of(idx_off(ci + 1), 8),
                                         idx_words)], nxt_idx, sem_i)

                def refills(g, b1, b2, s1, s2, local_nxt, first_next):
                    in_chunk = g + 2 < gpc
                    cross = jnp.logical_and(g + 2 >= gpc, ci + 1 < n_chunks)

                    def r1():
                        @pl.when(in_chunk)
                        def _():
                            gstart1(table, cur_idx, (g + 2) * _SEG, b1, s1)

                        @pl.when(cross)
                        def _():
                            if first_next:
                                pltpu.make_async_copy(
                                    idx_hbm.at[pl.ds(0, idx_words)],
                                    nxt_idx, sem_i).wait()
                            gstart1(table, nxt_idx, local_nxt, b1, s1)

                    def r2():
                        @pl.when(in_chunk)
                        def _():
                            gstart2(table, cur_idx, (g + 2) * _SEG, b2, s2)

                        @pl.when(cross)
                        def _():
                            gstart2(table, nxt_idx, local_nxt, b2, s2)

                    return r1, r2

                @pl.loop(0, gpc, step=2)
                def _g(j):
                    r1a, r2a = refills(j, buf_a1, buf_a2, sem_a1, sem_a2,
                                       0, True)
                    process(table, buf_a1, buf_a2, sem_a1, sem_a2, j,
                            stage_off, r1a, r2a)
                    r1b, r2b = refills(j + 1, buf_b1, buf_b2, sem_b1,
                                       sem_b2, _SEG, False)
                    process(table, buf_b1, buf_b2, sem_b1, sem_b2, j + 1,
                            stage_off, r1b, r2b)

                write_fn(ci)

            pltpu.sync_copy(
                idx_hbm.at[pl.ds(pl.multiple_of(idx_off(0), 8), idx_words)],
                idx_a)
            gstart1(table, idx_a, 0, buf_a1, sem_a1)
            gstart2(table, idx_a, 0, buf_a2, sem_a2)
            gstart1(table, idx_a, _SEG, buf_b1, sem_b1)
            gstart2(table, idx_a, _SEG, buf_b2, sem_b2)

            @pl.loop(0, n_chunks, step=2)
            def _cpair(ci):
                chunk_body(ci, idx_a, idx_b)
                chunk_body(ci + 1, idx_b, idx_a)

        def xs_write(ci):
            cb = b0 + ci * gpc
            for ch in range(_CH):
                pltpu.sync_copy(
                    stage_v.at[pl.ds(ch * gpc * _H1, gpc * _H1)],
                    xs_out.at[pl.ds(pl.multiple_of((ch * _B + cb) * _H1, 8),
                                    gpc * _H1)])

        run_phase(
            emb_hbm, x_hbm, spw // gpc,
            lambda ci: (b0 + ci * gpc) * _SEG,
            lambda g, s: s * (gpc * _H1) + g * _H1,
            xs_write)

        def cond_write(ci):
            cb = b0 + ci * 4 * gpc
            pltpu.sync_copy(
                stage_v,
                cond_out.at[pl.ds(pl.multiple_of(cb * _H1, 8), stage_words)])

        run_phase(
            embc_hbm, c_hbm, spw // (4 * gpc),
            lambda ci: (b0 + ci * 4 * gpc) * _L,
            lambda g, s: (g * 4 + s) * _H1,
            cond_write)

    return k(x_flat, cond_flat, embed, embed_condition)


def _tc_mlp(xs_raw, cond_raw, Wc, bc, W2, b2, W3, b3, W4, b4):
    bb = 512
    dn = (((1,), (1,)), ((), ()))

    def body(xs_ref, c_ref, wc_ref, bc_ref, w2_ref, b2_ref, w3_ref, b3_ref,
             w4_ref, b4_ref, out_ref):
        xs = jnp.maximum(xs_ref[...], 0.0)
        xsum = xs[0] + xs[1] + xs[2] + xs[3]
        c = jnp.maximum(c_ref[...], 0.0) + xsum
        c2 = lax.dot_general(c, wc_ref[...], dn,
                             preferred_element_type=jnp.float32)
        c2 = jnp.maximum(c2 + bc_ref[...], 0.0)
        outs = []
        for ch in range(_CH):
            h = lax.dot_general(xs[ch], w2_ref[...], dn,
                                preferred_element_type=jnp.float32)
            h = jnp.maximum(h + b2_ref[...] + c2, 0.0)
            h = lax.dot_general(h, w3_ref[...], dn,
                                preferred_element_type=jnp.float32)
            h = jnp.maximum(h + b3_ref[...], 0.0)
            o = lax.dot_general(h, w4_ref[...], dn,
                                preferred_element_type=jnp.float32)
            outs.append(o + b4_ref[...])
        out_ref[...] = jnp.concatenate(outs, axis=1)

    full = lambda shape: pl.BlockSpec(shape, lambda i: tuple(0 for _ in shape))
    return pl.pallas_call(
        body,
        grid=(_B // bb,),
        in_specs=[
            pl.BlockSpec((_CH, bb, _H1), lambda i: (0, i, 0)),
            pl.BlockSpec((bb, _H1), lambda i: (i, 0)),
            full((_H2, _H1)),
            full((1, _H2)),
            full((_H2, _H1)),
            full((1, _H2)),
            full((_H2, _H2)),
            full((1, _H2)),
            full((_OUT, _H2)),
            full((1, _OUT)),
        ],
        out_specs=pl.BlockSpec((bb, _CH * _OUT), lambda i: (i, 0)),
        out_shape=jax.ShapeDtypeStruct((_B, _CH * _OUT), jnp.float32),
    )(xs_raw, cond_raw, Wc, bc.reshape(1, _H2), W2, b2.reshape(1, _H2),
      W3, b3.reshape(1, _H2), W4, b4.reshape(1, _OUT))


def kernel(x, condition, embed, embed_condition, Wc, bc, W2, b2, W3, b3,
           W4, b4):
    x_flat = x.reshape(-1).astype(jnp.int32)
    cond_flat = condition.reshape(-1).astype(jnp.int32)
    xs_flat, cond_pool = _sc_gather_pool(x_flat, cond_flat, embed,
                                         embed_condition)
    xs_raw = xs_flat.reshape(_CH, _B, _H1)
    cond_raw = cond_pool.reshape(_B, _H1)
    out = _tc_mlp(xs_raw, cond_raw, Wc, bc, W2, b2, W3, b3, W4, b4)
    return out.reshape(_B, _CH, _OUT)


# speedup vs baseline: 1.2381x; 1.0124x over previous
import functools

import jax
import jax.numpy as jnp
from jax import lax
from jax.experimental import pallas as pl
from jax.experimental.pallas import tpu as pltpu
from jax.experimental.pallas import tpu_sc as plsc

_B = 4096
_CH = 4
_L = 50
_H1 = 256
_H2 = 32
_OUT = 5
_LANES = 16
_NCHK = _H1 // _LANES
_SEG = _CH * _L
_SP1 = 104


def _sc_gather_pool(x_flat, cond_flat, embed, embed_condition):
    mesh = plsc.VectorSubcoreMesh(
        core_axis_name="c", subcore_axis_name="s", num_cores=2, num_subcores=16
    )
    nw = mesh.num_cores * mesh.num_subcores
    spw = _B // nw
    gpc = 16
    idx_words = gpc * _SEG
    stage_words = gpc * _CH * _H1

    @functools.partial(
        pl.kernel,
        out_type=(
            jax.ShapeDtypeStruct((_CH * _B * _H1,), jnp.float32),
            jax.ShapeDtypeStruct((_B * _H1,), jnp.float32),
        ),
        mesh=mesh,
        scratch_types=[
            pltpu.VMEM((idx_words,), jnp.int32),
            pltpu.VMEM((idx_words,), jnp.int32),
            pltpu.VMEM((_SP1, _H1), jnp.float32),
            pltpu.VMEM((_SEG - _SP1, _H1), jnp.float32),
            pltpu.VMEM((_SP1, _H1), jnp.float32),
            pltpu.VMEM((_SEG - _SP1, _H1), jnp.float32),
            pltpu.VMEM((stage_words,), jnp.float32),
            pltpu.SemaphoreType.DMA,
            pltpu.SemaphoreType.DMA,
            pltpu.SemaphoreType.DMA,
            pltpu.SemaphoreType.DMA,
            pltpu.SemaphoreType.DMA,
        ],
    )
    def k(x_hbm, c_hbm, emb_hbm, embc_hbm, xs_out, cond_out,
          idx_a, idx_b, buf_a1, buf_a2, buf_b1, buf_b2, stage_v,
          sem_a1, sem_a2, sem_b1, sem_b2, sem_i):
        wid = lax.axis_index("s") * mesh.num_cores + lax.axis_index("c")
        b0 = wid * spw

        def gstart1(table, idx_v, goff, b1, s1):
            pltpu.async_copy(
                table.at[idx_v.at[pl.ds(pl.multiple_of(goff, 8), _SP1)]],
                b1, s1)

        def gstart2(table, idx_v, goff, b2, s2):
            pltpu.async_copy(
                table.at[idx_v.at[pl.ds(pl.multiple_of(goff + _SP1, 8),
                                        _SEG - _SP1)]], b2, s2)

        def gwait1(table, b1, s1):
            pltpu.make_async_copy(
                table.at[idx_a.at[pl.ds(0, _SP1)]], b1, s1).wait()

        def gwait2(table, b2, s2):
            pltpu.make_async_copy(
                table.at[idx_a.at[pl.ds(_SP1, _SEG - _SP1)]], b2, s2).wait()

        def accum_span(buf, row0, n, soff, cont):
            if cont:
                init = tuple(stage_v[pl.ds(soff + c * _LANES, _LANES)]
                             for c in range(_NCHK))
            else:
                init = tuple(jnp.zeros((_LANES,), jnp.float32)
                             for _ in range(_NCHK))

            @pl.loop(0, n, init_carry=init, unroll=2)
            def acc_loop(j, acc):
                r = row0 + j
                return tuple(acc[c] + buf[r, pl.ds(c * _LANES, _LANES)]
                             for c in range(_NCHK))

            for c in range(_NCHK):
                stage_v[pl.ds(soff + c * _LANES, _LANES)] = acc_loop[c]

        def process(table, b1, b2, s1, s2, g, stage_off, refill1, refill2):
            gwait1(table, b1, s1)
            accum_span(b1, 0, _L, stage_off(g, 0), False)
            accum_span(b1, _L, _L, stage_off(g, 1), False)
            accum_span(b1, 2 * _L, _SP1 - 2 * _L, stage_off(g, 2), False)
            refill1()
            gwait2(table, b2, s2)
            accum_span(b2, 0, 3 * _L - _SP1, stage_off(g, 2), True)
            accum_span(b2, 3 * _L - _SP1, _L, stage_off(g, 3), False)
            refill2()

        def run_phase(table, idx_hbm, n_chunks, idx_off, stage_off, write_fn):
            def chunk_body(ci, cur_idx, nxt_idx):
                @pl.when(ci + 1 < n_chunks)
                def _():
                    pltpu.async_copy(
                        idx_hbm.at[pl.ds(pl.multiple_of(idx_off(ci + 1), 8),
                                         idx_words)], nxt_idx, sem_i)

                def refills(g, b1, b2, s1, s2, local_nxt, first_next):
                    in_chunk = g + 2 < gpc
                    cross = jnp.logical_and(g + 2 >= gpc, ci + 1 < n_chunks)

                    def r1():
                        @pl.when(in_chunk)
                        def _():
                            gstart1(table, cur_idx, (g + 2) * _SEG, b1, s1)

                        @pl.when(cross)
                        def _():
                            if first_next:
                                pltpu.make_async_copy(
                                    idx_hbm.at[pl.ds(0, idx_words)],
                                    nxt_idx, sem_i).wait()
                            gstart1(table, nxt_idx, local_nxt, b1, s1)

                    def r2():
                        @pl.when(in_chunk)
                        def _():
                            gstart2(table, cur_idx, (g + 2) * _SEG, b2, s2)

                        @pl.when(cross)
                        def _():
                            gstart2(table, nxt_idx, local_nxt, b2, s2)

                    return r1, r2

                @pl.loop(0, gpc, step=2)
                def _g(j):
                    r1a, r2a = refills(j, buf_a1, buf_a2, sem_a1, sem_a2,
                                       0, True)
                    process(table, buf_a1, buf_a2, sem_a1, sem_a2, j,
                            stage_off, r1a, r2a)
                    r1b, r2b = refills(j + 1, buf_b1, buf_b2, sem_b1,
                                       sem_b2, _SEG, False)
                    process(table, buf_b1, buf_b2, sem_b1, sem_b2, j + 1,
                            stage_off, r1b, r2b)

                write_fn(ci)

            pltpu.sync_copy(
                idx_hbm.at[pl.ds(pl.multiple_of(idx_off(0), 8), idx_words)],
                idx_a)
            gstart1(table, idx_a, 0, buf_a1, sem_a1)
            gstart2(table, idx_a, 0, buf_a2, sem_a2)
            gstart1(table, idx_a, _SEG, buf_b1, sem_b1)
            gstart2(table, idx_a, _SEG, buf_b2, sem_b2)

            @pl.loop(0, n_chunks, step=2)
            def _cpair(ci):
                chunk_body(ci, idx_a, idx_b)
                chunk_body(ci + 1, idx_b, idx_a)

        def xs_write(ci):
            cb = b0 + ci * gpc
            for ch in range(_CH):
                pltpu.sync_copy(
                    stage_v.at[pl.ds(ch * gpc * _H1, gpc * _H1)],
                    xs_out.at[pl.ds(pl.multiple_of((ch * _B + cb) * _H1, 8),
                                    gpc * _H1)])

        run_phase(
            emb_hbm, x_hbm, spw // gpc,
            lambda ci: (b0 + ci * gpc) * _SEG,
            lambda g, s: s * (gpc * _H1) + g * _H1,
            xs_write)

        def cond_write(ci):
            cb = b0 + ci * 4 * gpc
            pltpu.sync_copy(
                stage_v,
                cond_out.at[pl.ds(pl.multiple_of(cb * _H1, 8), stage_words)])

        run_phase(
            embc_hbm, c_hbm, spw // (4 * gpc),
            lambda ci: (b0 + ci * 4 * gpc) * _L,
            lambda g, s: (g * 4 + s) * _H1,
            cond_write)

    return k(x_flat, cond_flat, embed, embed_condition)


def _tc_mlp(xs_raw, cond_raw, Wc, bc, W2, b2, W3, b3, W4, b4):
    bb = 512
    dn = (((1,), (1,)), ((), ()))

    def body(xs_ref, c_ref, wc_ref, bc_ref, w2_ref, b2_ref, w3_ref, b3_ref,
             w4_ref, b4_ref, out_ref):
        xs = jnp.maximum(xs_ref[...], 0.0)
        xsum = xs[0] + xs[1] + xs[2] + xs[3]
        c = jnp.maximum(c_ref[...], 0.0) + xsum
        c2 = lax.dot_general(c, wc_ref[...], dn,
                             preferred_element_type=jnp.float32)
        c2 = jnp.maximum(c2 + bc_ref[...], 0.0)
        outs = []
        for ch in range(_CH):
            h = lax.dot_general(xs[ch], w2_ref[...], dn,
                                preferred_element_type=jnp.float32)
            h = jnp.maximum(h + b2_ref[...] + c2, 0.0)
            h = lax.dot_general(h, w3_ref[...], dn,
                                preferred_element_type=jnp.float32)
            h = jnp.maximum(h + b3_ref[...], 0.0)
            o = lax.dot_general(h, w4_ref[...], dn,
                                preferred_element_type=jnp.float32)
            outs.append(o + b4_ref[...])
        out_ref[...] = jnp.concatenate(outs, axis=1)

    full = lambda shape: pl.BlockSpec(shape, lambda i: tuple(0 for _ in shape))
    return pl.pallas_call(
        body,
        grid=(_B // bb,),
        in_specs=[
            pl.BlockSpec((_CH, bb, _H1), lambda i: (0, i, 0)),
            pl.BlockSpec((bb, _H1), lambda i: (i, 0)),
            full((_H2, _H1)),
            full((1, _H2)),
            full((_H2, _H1)),
            full((1, _H2)),
            full((_H2, _H2)),
            full((1, _H2)),
            full((_OUT, _H2)),
            full((1, _OUT)),
        ],
        out_specs=pl.BlockSpec((bb, _CH * _OUT), lambda i: (i, 0)),
        out_shape=jax.ShapeDtypeStruct((_B, _CH * _OUT), jnp.float32),
    )(xs_raw, cond_raw, Wc, bc.reshape(1, _H2), W2, b2.reshape(1, _H2),
      W3, b3.reshape(1, _H2), W4, b4.reshape(1, _OUT))


def kernel(x, condition, embed, embed_condition, Wc, bc, W2, b2, W3, b3,
           W4, b4):
    x_flat = x.reshape(-1).astype(jnp.int32)
    cond_flat = condition.reshape(-1).astype(jnp.int32)
    xs_flat, cond_pool = _sc_gather_pool(x_flat, cond_flat, embed,
                                         embed_condition)
    xs_raw = xs_flat.reshape(_CH, _B, _H1)
    cond_raw = cond_pool.reshape(_B, _H1)
    out = _tc_mlp(xs_raw, cond_raw, Wc, bc, W2, b2, W3, b3, W4, b4)
    return out.reshape(_B, _CH, _OUT)
